# baseline (speedup 1.0000x reference)
import functools

import jax
import jax.numpy as jnp
from jax import lax
from jax.experimental import pallas as pl
from jax.experimental.pallas import tpu as pltpu
from jax.experimental.pallas import tpu_sc as plsc

NC = 2
NSUB = 16
NW = NC * NSUB
L = 16

B = 16384
D = 64
NNEG = 5
VOCAB = 1000000
HALFV = VOCAB // 2
W = 2 * D
BPW = B // NW
CH = 64
NCHUNK = BPW // CH
GRP = CH // L
DV = D // L
NSC = 1 + NNEG


def _splat(ref, idx):
    return plsc.load_gather(ref, [jnp.full((L,), idx, jnp.int32)])


def _sc_body(uidx, uoff, vidx, voff, nidx, noff, up, vp, out,
             uidx_v, uoff_v, vidx_v, voff_v, nidx_v, noff_v,
             urows_v, vrows_v, nrows_v, scores_v, sem):
    wid = lax.axis_index("s") * NC + lax.axis_index("c")
    base = wid * BPW
    pltpu.sync_copy(uidx.at[pl.ds(base, BPW)], uidx_v)
    pltpu.sync_copy(uoff.at[pl.ds(base, BPW)], uoff_v)
    pltpu.sync_copy(vidx.at[pl.ds(base, BPW)], vidx_v)
    pltpu.sync_copy(voff.at[pl.ds(base, BPW)], voff_v)
    pltpu.sync_copy(nidx.at[pl.ds(base * NNEG, BPW * NNEG)], nidx_v)
    pltpu.sync_copy(noff.at[pl.ds(base * NNEG, BPW * NNEG)], noff_v)

    lane = lax.iota(jnp.int32, L)

    def chunk_body(c, _):
        cps = [
            pltpu.async_copy(up.at[uidx_v.at[pl.ds(c * CH, CH)]], urows_v, sem),
            pltpu.async_copy(vp.at[vidx_v.at[pl.ds(c * CH, CH)]], vrows_v, sem),
        ]
        for q in range(NNEG):
            cps.append(pltpu.async_copy(
                vp.at[nidx_v.at[pl.ds(c * (CH * NNEG) + q * CH, CH)]],
                nrows_v.at[pl.ds(q * CH, CH)], sem))
        for cp in cps:
            cp.wait()

        def group_body(g, _):
            accs = [jnp.zeros((L,), jnp.float32) for _ in range(NSC)]
            for j in range(L):
                e = g * L + j
                uo = _splat(uoff_v, c * CH + e)
                vo = _splat(voff_v, c * CH + e)
                us = [plsc.load_gather(urows_v, [jnp.full((L,), e, jnp.int32),
                                                 uo + (k * L + lane)])
                      for k in range(DV)]
                vs = [plsc.load_gather(vrows_v, [jnp.full((L,), e, jnp.int32),
                                                 vo + (k * L + lane)])
                      for k in range(DV)]
                s = jnp.sum(sum(u * v for u, v in zip(us, vs)))
                accs[0] = jnp.where(lane == j, s, accs[0])
                for q in range(NNEG):
                    p = e * NNEG + q
                    no = _splat(noff_v, c * (CH * NNEG) + p)
                    ns = [plsc.load_gather(nrows_v,
                                           [jnp.full((L,), p, jnp.int32),
                                            no + (k * L + lane)])
                          for k in range(DV)]
                    s = jnp.sum(sum(u * n for u, n in zip(us, ns)))
                    accs[1 + q] = jnp.where(lane == j, s, accs[1 + q])
            for r in range(NSC):
                scores_v[pl.ds(r * BPW + c * CH + g * L, L)] = accs[r]
            return 0

        lax.fori_loop(0, GRP, group_body, 0)
        return 0

    lax.fori_loop(0, NCHUNK, chunk_body, 0)
    pltpu.sync_copy(scores_v, out.at[pl.ds(base * NSC, BPW * NSC)])


@jax.jit
def _sc_scores(uidx, uoff, vidx, voff, nidx, noff, up, vp):
    mesh = plsc.VectorSubcoreMesh(core_axis_name="c", subcore_axis_name="s")
    return pl.kernel(
        _sc_body,
        out_type=jax.ShapeDtypeStruct((B * NSC,), jnp.float32),
        mesh=mesh,
        compiler_params=pltpu.CompilerParams(
            needs_layout_passes=False, use_tc_tiling_on_sc=True),
        scratch_types=[
            pltpu.VMEM((BPW,), jnp.int32),
            pltpu.VMEM((BPW,), jnp.int32),
            pltpu.VMEM((BPW,), jnp.int32),
            pltpu.VMEM((BPW,), jnp.int32),
            pltpu.VMEM((BPW * NNEG,), jnp.int32),
            pltpu.VMEM((BPW * NNEG,), jnp.int32),
            pltpu.VMEM((CH, W), jnp.float32),
            pltpu.VMEM((CH, W), jnp.float32),
            pltpu.VMEM((CH * NNEG, W), jnp.float32),
            pltpu.VMEM((BPW * NSC,), jnp.float32),
            pltpu.SemaphoreType.DMA,
        ],
    )(uidx, uoff, vidx, voff, nidx, noff, up, vp)


def _loss_body(scores_ref, out_ref):
    s = scores_ref[...]
    r = lax.broadcasted_iota(jnp.int32, s.shape, 1)
    x = jnp.where(r == 0, s, -s)
    ls = jax.nn.log_sigmoid(x)
    out_ref[...] = jnp.full((1, 1), -jnp.sum(ls) / B, jnp.float32)


@jax.jit
def _loss(scores):
    out = pl.pallas_call(
        _loss_body,
        out_shape=jax.ShapeDtypeStruct((1, 1), jnp.float32),
    )(scores.reshape(NW, NSC, BPW))
    return out[0, 0]


def kernel(pos_u, pos_v, neg_v, u_weight, v_weight):
    up = u_weight[:VOCAB].reshape(HALFV, W)
    vp = v_weight[:VOCAB].reshape(HALFV, W)
    negf = neg_v.reshape(-1)
    scores = _sc_scores(
        lax.shift_right_logical(pos_u, 1), (pos_u & 1) * D,
        lax.shift_right_logical(pos_v, 1), (pos_v & 1) * D,
        lax.shift_right_logical(negf, 1), (negf & 1) * D,
        up, vp)
    return _loss(scores)
